# Initial kernel scaffold; baseline (speedup 1.0000x reference)
import jax
import jax.numpy as jnp
from jax.experimental import pallas as pl


def kernel(x, edge_index, edges_coarse, pseudos, hexes, params):
    raise NotImplementedError("write your pallas kernel here")


# speedup vs baseline: 3.7416x; 3.7416x over previous
import functools

import jax
import jax.numpy as jnp
from jax import lax
from jax.experimental import pallas as pl
from jax.experimental.pallas import tpu as pltpu
from jax.experimental.pallas import tpu_sc as plsc

KM = 3
NC, NS, LN = 2, 16, 16
NW = NC * NS

NV = [40962, 10242, 2562, 642, 162]
NPAD = [41472, 10752, 3072, 1024, 512]
EV = [245760, 61440, 15360, 3840, 960]
EPAD = [245760, 61440, 16384, 4096, 4096]

SPECS = [(4, 32), (32, 64), (64, 128), (128, 256), (256, 256),
         (384, 128), (192, 64), (96, 32), (64, 21)]
SPLITS = {5: (256, 128), 6: (128, 64), 7: (64, 32), 8: (32, 32)}
LEVEL_OF = [0, 1, 2, 3, 4, 3, 2, 1, 0]
BM = 512

_MM_CACHE = {}
_FIN_CACHE = {}
_SM_CACHE = {}
_GMM_CACHE = {}
_CNT_CACHE = {}
_POOL_CACHE = {}
_UNPOOL_CACHE = {}


def _ceil_to(v, m):
    return (v + m - 1) // m * m


def _mm_call(np_rows, ca, cb, op):
    key = (np_rows, ca, cb, op)
    if key in _MM_CACHE:
        return _MM_CACHE[key]
    two = cb > 0

    if two:
        def body(a_ref, b_ref, ga_ref, gb_ref, ra_ref, rb_ref, bias_ref,
                 y_ref, r_ref):
            y = jnp.dot(a_ref[...], ga_ref[...],
                        preferred_element_type=jnp.float32)
            y += jnp.dot(b_ref[...], gb_ref[...],
                         preferred_element_type=jnp.float32)
            r = jnp.dot(a_ref[...], ra_ref[...],
                        preferred_element_type=jnp.float32)
            r += jnp.dot(b_ref[...], rb_ref[...],
                         preferred_element_type=jnp.float32)
            y_ref[...] = y
            r_ref[...] = r + bias_ref[0:1, :]
        in_specs = [
            pl.BlockSpec((BM, ca), lambda i: (i, 0)),
            pl.BlockSpec((BM, cb), lambda i: (i, 0)),
            pl.BlockSpec((ca, 3 * op), lambda i: (0, 0)),
            pl.BlockSpec((cb, 3 * op), lambda i: (0, 0)),
            pl.BlockSpec((ca, op), lambda i: (0, 0)),
            pl.BlockSpec((cb, op), lambda i: (0, 0)),
            pl.BlockSpec((8, op), lambda i: (0, 0)),
        ]
    else:
        def body(a_ref, ga_ref, ra_ref, bias_ref, y_ref, r_ref):
            y_ref[...] = jnp.dot(a_ref[...], ga_ref[...],
                                 preferred_element_type=jnp.float32)
            r = jnp.dot(a_ref[...], ra_ref[...],
                        preferred_element_type=jnp.float32)
            r_ref[...] = r + bias_ref[0:1, :]
        in_specs = [
            pl.BlockSpec((BM, ca), lambda i: (i, 0)),
            pl.BlockSpec((ca, 3 * op), lambda i: (0, 0)),
            pl.BlockSpec((ca, op), lambda i: (0, 0)),
            pl.BlockSpec((8, op), lambda i: (0, 0)),
        ]

    fn = pl.pallas_call(
        body,
        grid=(np_rows // BM,),
        in_specs=in_specs,
        out_specs=[pl.BlockSpec((BM, 3 * op), lambda i: (i, 0)),
                   pl.BlockSpec((BM, op), lambda i: (i, 0))],
        out_shape=[jax.ShapeDtypeStruct((np_rows, 3 * op), jnp.float32),
                   jax.ShapeDtypeStruct((np_rows, op), jnp.float32)],
    )
    _MM_CACHE[key] = fn
    return fn


def _fin_call(np_rows, op):
    key = (np_rows, op)
    if key in _FIN_CACHE:
        return _FIN_CACHE[key]

    def body(acc_ref, cnt_ref, r_ref, o_ref):
        s = acc_ref[0] + acc_ref[1]
        c = cnt_ref[0, :, 0:1] + cnt_ref[1, :, 0:1]
        o = s / jnp.maximum(c, 1.0) + r_ref[...]
        o_ref[...] = jnp.maximum(o, 0.0)

    fn = pl.pallas_call(
        body,
        grid=(np_rows // BM,),
        in_specs=[
            pl.BlockSpec((2, BM, op), lambda i: (0, i, 0)),
            pl.BlockSpec((2, BM, 16), lambda i: (0, i, 0)),
            pl.BlockSpec((BM, op), lambda i: (i, 0)),
        ],
        out_specs=pl.BlockSpec((BM, op), lambda i: (i, 0)),
        out_shape=jax.ShapeDtypeStruct((np_rows, op), jnp.float32),
    )
    _FIN_CACHE[key] = fn
    return fn


def _softmax_call(np_rows, op, real_cols):
    key = (np_rows, op, real_cols)
    if key in _SM_CACHE:
        return _SM_CACHE[key]

    def body(acc_ref, cnt_ref, r_ref, o_ref):
        s = acc_ref[0] + acc_ref[1]
        c = cnt_ref[0, :, 0:1] + cnt_ref[1, :, 0:1]
        z = s / jnp.maximum(c, 1.0) + r_ref[...]
        col = lax.broadcasted_iota(jnp.int32, z.shape, 1)
        mask = col < real_cols
        zm = jnp.where(mask, z, -jnp.inf)
        m = jnp.max(zm, axis=1, keepdims=True)
        e = jnp.where(mask, jnp.exp(z - m), 0.0)
        o_ref[...] = e / jnp.sum(e, axis=1, keepdims=True)

    fn = pl.pallas_call(
        body,
        grid=(np_rows // BM,),
        in_specs=[
            pl.BlockSpec((2, BM, op), lambda i: (0, i, 0)),
            pl.BlockSpec((2, BM, 16), lambda i: (0, i, 0)),
            pl.BlockSpec((BM, op), lambda i: (i, 0)),
        ],
        out_specs=pl.BlockSpec((BM, op), lambda i: (i, 0)),
        out_shape=jax.ShapeDtypeStruct((np_rows, op), jnp.float32),
    )
    _SM_CACHE[key] = fn
    return fn


def _mesh():
    return plsc.VectorSubcoreMesh(core_axis_name="c", subcore_axis_name="s")


_SC_PARAMS = pltpu.CompilerParams(use_tc_tiling_on_sc=False, needs_layout_passes=False)


def _lane_consts():
    return [jnp.full((LN, 1), i, dtype=jnp.int32) for i in range(LN)]


_GDN = lax.GatherDimensionNumbers(
    offset_dims=(), collapsed_slice_dims=(0,), start_index_map=(0,))


def _bcast_lane(vec, lane_const):
    return lax.gather(vec, lane_const, _GDN, (1,),
                      mode=lax.GatherScatterMode.PROMISE_IN_BOUNDS)


def _gmm_sc_call(npad, epad, op, bsz):
    key = (npad, epad, op, bsz)
    if key in _GMM_CACHE:
        return _GMM_CACHE[key]
    cols = 3 * op
    nj = op // LN
    per_tile = epad // NW
    n_chunks = per_tile // bsz
    zrows = 32
    rpt_sc = npad // NS
    nz = rpt_sc // zrows

    @functools.partial(
        pl.kernel,
        out_type=jax.ShapeDtypeStruct((NC, npad, op), jnp.float32),
        mesh=_mesh(),
        compiler_params=_SC_PARAMS,
        scratch_types=[
            pltpu.VMEM((bsz,), jnp.int32),
            pltpu.VMEM((bsz,), jnp.int32),
            pltpu.VMEM((2, bsz), jnp.float32),
            pltpu.VMEM((KM, bsz), jnp.float32),
            pltpu.VMEM((bsz, cols), jnp.float32),
            pltpu.VMEM((bsz, op), jnp.float32),
            pltpu.VMEM((KM, 2, LN), jnp.float32),
            pltpu.VMEM((KM, 2, LN), jnp.float32),
            pltpu.VMEM((zrows, op), jnp.float32),
            pltpu.VMEM_SHARED((npad, op), jnp.float32),
            pltpu.SemaphoreType.DMA,
        ],
    )
    def k(ei, psT, s2e_h, mu_h, ytab, out_h,
          src_v, dst_v, ps_v, w_v, rows_v, msg_v, s2_v, mu_v, zero_v,
          acc_sh, sem):
        cid = lax.axis_index("c")
        sid = lax.axis_index("s")
        wid = sid * NC + cid
        lanes = _lane_consts()
        zv = jnp.zeros((LN,), jnp.float32)

        for r in range(zrows):
            for j in range(nj):
                zero_v[r, pl.ds(j * LN, LN)] = zv
        zbase = sid * rpt_sc

        @pl.loop(0, nz)
        def _zero(i):
            pltpu.sync_copy(zero_v, acc_sh.at[pl.ds(zbase + i * zrows, zrows)])

        pltpu.sync_copy(s2e_h, s2_v)
        pltpu.sync_copy(mu_h, mu_v)
        plsc.subcore_barrier()

        @pl.loop(0, n_chunks)
        def _chunk(i):
            base = (wid * n_chunks + i) * bsz
            pltpu.sync_copy(ei.at[0, pl.ds(base, bsz)], src_v)
            pltpu.sync_copy(ei.at[1, pl.ds(base, bsz)], dst_v)
            pltpu.sync_copy(psT.at[0, pl.ds(base, bsz)], ps_v.at[0])
            pltpu.sync_copy(psT.at[1, pl.ds(base, bsz)], ps_v.at[1])
            pltpu.async_copy(ytab.at[src_v], rows_v, sem).wait()
            for t in range(bsz // LN):
                p0 = ps_v[0, pl.ds(t * LN, LN)]
                p1 = ps_v[1, pl.ds(t * LN, LN)]
                for kk in range(KM):
                    d0 = p0 - mu_v[kk, 0]
                    d1 = p1 - mu_v[kk, 1]
                    q = d0 * d0 / s2_v[kk, 0] + d1 * d1 / s2_v[kk, 1]
                    w_v[kk, pl.ds(t * LN, LN)] = jnp.exp(q * (-0.5))
            for t in range(bsz // LN):
                w0 = w_v[0, pl.ds(t * LN, LN)]
                w1 = w_v[1, pl.ds(t * LN, LN)]
                w2 = w_v[2, pl.ds(t * LN, LN)]
                for ln in range(LN):
                    e = t * LN + ln
                    b0 = _bcast_lane(w0, lanes[ln])
                    b1 = _bcast_lane(w1, lanes[ln])
                    b2 = _bcast_lane(w2, lanes[ln])
                    for j in range(nj):
                        m = (b0 * rows_v[e, pl.ds(j * LN, LN)]
                             + b1 * rows_v[e, pl.ds(op + j * LN, LN)]
                             + b2 * rows_v[e, pl.ds(2 * op + j * LN, LN)])
                        msg_v[e, pl.ds(j * LN, LN)] = m
            pltpu.sync_copy(msg_v, acc_sh.at[dst_v], add=True)

        plsc.subcore_barrier()

        @pl.loop(0, nz)
        def _out(i):
            b = zbase + i * zrows
            pltpu.sync_copy(acc_sh.at[pl.ds(b, zrows)],
                            out_h.at[cid, pl.ds(b, zrows)])

    _GMM_CACHE[key] = k
    return k


def _cnt_sc_call(npad, epad):
    key = (npad, epad)
    if key in _CNT_CACHE:
        return _CNT_CACHE[key]
    bsz = 128
    per_tile = epad // NW
    n_chunks = per_tile // bsz
    zrows = 32
    rpt_sc = npad // NS
    nz = rpt_sc // zrows

    @functools.partial(
        pl.kernel,
        out_type=jax.ShapeDtypeStruct((NC, npad, 16), jnp.float32),
        mesh=_mesh(),
        compiler_params=_SC_PARAMS,
        scratch_types=[
            pltpu.VMEM((bsz,), jnp.int32),
            pltpu.VMEM((bsz, 16), jnp.float32),
            pltpu.VMEM((zrows, 16), jnp.float32),
            pltpu.VMEM_SHARED((npad, 16), jnp.float32),
        ],
    )
    def k(ei, out_h, dst_v, ones_v, zero_v, acc_sh):
        cid = lax.axis_index("c")
        sid = lax.axis_index("s")
        wid = sid * NC + cid
        zv = jnp.zeros((LN,), jnp.float32)
        ov = jnp.where(lax.iota(jnp.int32, LN) == 0, 1.0, 0.0)
        for r in range(zrows):
            zero_v[r, pl.ds(0, LN)] = zv
        for r in range(bsz):
            ones_v[r, pl.ds(0, LN)] = ov
        zbase = sid * rpt_sc

        @pl.loop(0, nz)
        def _zero(i):
            pltpu.sync_copy(zero_v, acc_sh.at[pl.ds(zbase + i * zrows, zrows)])

        plsc.subcore_barrier()

        @pl.loop(0, n_chunks)
        def _chunk(i):
            base = (wid * n_chunks + i) * bsz
            pltpu.sync_copy(ei.at[1, pl.ds(base, bsz)], dst_v)
            pltpu.sync_copy(ones_v, acc_sh.at[dst_v], add=True)

        plsc.subcore_barrier()

        @pl.loop(0, nz)
        def _out(i):
            b = zbase + i * zrows
            pltpu.sync_copy(acc_sh.at[pl.ds(b, zrows)],
                            out_h.at[cid, pl.ds(b, zrows)])

    _CNT_CACHE[key] = k
    return k


def _pool_sc_call(np_coarse, cch, l_real, big):
    key = (np_coarse, cch, l_real)
    if key in _POOL_CACHE:
        return _POOL_CACHE[key]
    nj = cch // LN
    rpt = np_coarse // NW

    @functools.partial(
        pl.kernel,
        out_type=(jax.ShapeDtypeStruct((np_coarse, cch), jnp.float32),
                  jax.ShapeDtypeStruct((np_coarse, cch), jnp.int32)),
        mesh=_mesh(),
        compiler_params=_SC_PARAMS,
        scratch_types=[
            pltpu.VMEM((7, rpt), jnp.int32),
            pltpu.VMEM((7, LN, cch), jnp.float32),
            pltpu.VMEM((LN, cch), jnp.float32),
            pltpu.VMEM((LN, cch), jnp.int32),
            pltpu.SemaphoreType.DMA,
        ],
    )
    def k(x_h, hexT, vals_h, idx_h, hx_v, xg_v, vb_v, ib_v, sem):
        cid = lax.axis_index("c")
        sid = lax.axis_index("s")
        wid = sid * NC + cid
        lanes = _lane_consts()
        bigv = jnp.full((LN,), big, dtype=jnp.int32)
        row0 = wid * rpt
        for j in range(7):
            pltpu.sync_copy(hexT.at[j, pl.ds(row0, rpt)], hx_v.at[j])

        @pl.loop(0, rpt // LN)
        def _chunk(ch):
            for j in range(7):
                pltpu.async_copy(x_h.at[hx_v.at[j, pl.ds(ch * LN, LN)]],
                                 xg_v.at[j], sem).wait()
            base = row0 + ch * LN
            for r in range(LN):
                hv0 = hx_v[0, pl.ds(ch * LN, LN)]
                bi0 = _bcast_lane(hv0, lanes[r])
                bv = [xg_v[0, r, pl.ds(j * LN, LN)] for j in range(nj)]
                bi = [bi0 for _ in range(nj)]
                for j in range(1, 7):
                    hv = hx_v[j, pl.ds(ch * LN, LN)]
                    bj = _bcast_lane(hv, lanes[r])
                    for jc in range(nj):
                        v = xg_v[j, r, pl.ds(jc * LN, LN)]
                        m = v > bv[jc]
                        bv[jc] = jnp.where(m, v, bv[jc])
                        bi[jc] = jnp.where(m, bj, bi[jc])
                valid = jnp.full((LN,), base + r < l_real)
                for jc in range(nj):
                    vb_v[r, pl.ds(jc * LN, LN)] = bv[jc]
                    ib_v[r, pl.ds(jc * LN, LN)] = jnp.where(valid, bi[jc],
                                                            bigv)
            pltpu.sync_copy(vb_v, vals_h.at[pl.ds(base, LN)])
            pltpu.sync_copy(ib_v, idx_h.at[pl.ds(base, LN)])

    _POOL_CACHE[key] = k
    return k


def _unpool_sc_call(np_coarse, np_fine, cch, bu):
    key = (np_coarse, np_fine, cch, bu)
    if key in _UNPOOL_CACHE:
        return _UNPOOL_CACHE[key]
    nj = cch // LN
    rt = np_fine // NW
    n_chunks = np_coarse // bu

    @functools.partial(
        pl.kernel,
        out_type=jax.ShapeDtypeStruct((np_fine, cch), jnp.float32),
        mesh=_mesh(),
        compiler_params=_SC_PARAMS,
        scratch_types=[
            pltpu.VMEM((bu, cch), jnp.float32),
            pltpu.VMEM((bu, cch), jnp.int32),
            pltpu.VMEM((rt, cch), jnp.float32),
        ],
    )
    def k(vals_h, idx_h, y_h, v_v, i_v, y_v, ):
        cid = lax.axis_index("c")
        sid = lax.axis_index("s")
        wid = sid * NC + cid
        base = wid * rt
        basev = jnp.full((LN,), 0, dtype=jnp.int32) + base
        rtv = jnp.full((LN,), rt, dtype=jnp.int32)
        zv = jnp.zeros((LN,), jnp.float32)
        cols = [lax.iota(jnp.int32, LN) + (jc * LN) for jc in range(nj)]

        @pl.loop(0, rt)
        def _zero(r):
            for j in range(nj):
                y_v[r, pl.ds(j * LN, LN)] = zv

        @pl.loop(0, n_chunks)
        def _chunk(i):
            pltpu.sync_copy(vals_h.at[pl.ds(i * bu, bu)], v_v)
            pltpu.sync_copy(idx_h.at[pl.ds(i * bu, bu)], i_v)
            for r in range(bu):
                for jc in range(nj):
                    iv = i_v[r, pl.ds(jc * LN, LN)]
                    vv = v_v[r, pl.ds(jc * LN, LN)]
                    lr = iv - basev
                    m = lr.astype(jnp.uint32) < rtv.astype(jnp.uint32)
                    plsc.store_scatter(y_v, [lr, cols[jc]], vv, mask=m)

        pltpu.sync_copy(y_v, y_h.at[pl.ds(base, rt)])

    _UNPOOL_CACHE[key] = k
    return k


def _pad_rows(a, rows):
    return jnp.pad(a, ((0, rows - a.shape[0]), (0, 0)))


def _prep_conv(p, ci):
    g, mu, sigma, root, bias = p
    in_c, out_c = SPECS[ci]
    op = _ceil_to(out_c, 32)
    g3 = g.reshape(in_c, KM, out_c)
    g3 = jnp.pad(g3, ((0, 0), (0, 0), (0, op - out_c)))
    gp = g3.reshape(in_c, KM * op)
    rootp = jnp.pad(root, ((0, 0), (0, op - out_c)))
    biasp = jnp.pad(bias, (0, op - out_c))
    bias8 = jnp.broadcast_to(biasp[None, :], (8, op))
    s2e = jnp.broadcast_to((sigma ** 2 + 1e-16)[:, :, None], (KM, 2, LN))
    mub = jnp.broadcast_to(mu[:, :, None], (KM, 2, LN))
    if ci == 0:
        gp = jnp.pad(gp, ((0, 4), (0, 0)))
        rootp = jnp.pad(rootp, ((0, 4), (0, 0)))
    return gp, rootp, bias8, s2e, mub, op


def _gmm_bsz(op):
    return {32: 128, 64: 128, 128: 64, 256: 32}[op]


def kernel(x, edge_index, edges_coarse, pseudos, hexes, params):
    eis, psTs = [], []
    for lv in range(5):
        ei = edge_index if lv == 0 else edges_coarse[lv - 1]
        ep = EPAD[lv]
        pad = ep - EV[lv]
        src = jnp.pad(ei[0], (0, pad))
        dst = jnp.pad(ei[1], (0, pad), constant_values=NV[lv])
        eis.append(jnp.stack([src, dst]))
        psTs.append(jnp.pad(pseudos[lv].T, ((0, 0), (0, pad))))
    hexTs = []
    for lv in range(4):
        lc = NV[lv + 1]
        hexT = hexes[lv][:lc].T
        hexTs.append(jnp.pad(hexT, ((0, 0), (0, NPAD[lv + 1] - lc))))
    prep = [_prep_conv(params[i], i) for i in range(9)]

    cnts = [_cnt_sc_call(NPAD[lv], EPAD[lv])(eis[lv]) for lv in range(5)]

    def gmm(ci, xa, xb, relu):
        lv = LEVEL_OF[ci]
        gp, rootp, bias8, s2e, mub, op = prep[ci]
        npr = NPAD[lv]
        if xb is None:
            y, r = _mm_call(npr, xa.shape[1], 0, op)(xa, gp, rootp, bias8)
        else:
            ca, cb = SPLITS[ci]
            y, r = _mm_call(npr, ca, cb, op)(
                xa, xb, gp[:ca], gp[ca:], rootp[:ca], rootp[ca:], bias8)
        bsz = _gmm_bsz(op)
        acc = _gmm_sc_call(npr, EPAD[lv], op, bsz)(
            eis[lv], psTs[lv], s2e, mub, y)
        if relu:
            return _fin_call(npr, op)(acc, cnts[lv], r)
        return _softmax_call(npr, op, SPECS[ci][1])(acc, cnts[lv], r)

    def pool(lv, xv):
        cch = xv.shape[1]
        return _pool_sc_call(NPAD[lv + 1], cch, NV[lv + 1], 1 << 28)(
            xv, hexTs[lv])

    def unpool(lv, vals, idx):
        cch = vals.shape[1]
        bu = {256: 32, 128: 64, 64: 128, 32: 128}[cch]
        return _unpool_sc_call(NPAD[lv + 1], NPAD[lv], cch, bu)(vals, idx)

    x0in = _pad_rows(jnp.pad(x, ((0, 0), (0, 4))), NPAD[0])
    x0 = gmm(0, x0in, None, True)
    x1, i1 = pool(0, x0)
    h = gmm(1, x1, None, True)
    x2, i2 = pool(1, h)
    h = gmm(2, x2, None, True)
    x3, i3 = pool(2, h)
    h = gmm(3, x3, None, True)
    x4, i4 = pool(3, h)
    h = gmm(4, x4, None, True)
    h = unpool(3, h, i4)
    h = gmm(5, h, x3, True)
    h = unpool(2, h, i3)
    h = gmm(6, h, x2, True)
    h = unpool(1, h, i2)
    h = gmm(7, h, x1, True)
    h = unpool(0, h, i1)
    out = gmm(8, h, x0, False)
    return out[:NV[0], :SPECS[8][1]]


# speedup vs baseline: 4.6598x; 1.2454x over previous
import functools

import jax
import jax.numpy as jnp
from jax import lax
from jax.experimental import pallas as pl
from jax.experimental.pallas import tpu as pltpu
from jax.experimental.pallas import tpu_sc as plsc

KM = 3
NC, NS, LN = 2, 16, 16
NW = NC * NS

NV = [40962, 10242, 2562, 642, 162]
NPAD = [41472, 10752, 3072, 1024, 512]
EV = [245760, 61440, 15360, 3840, 960]
EPAD = [245760, 61440, 16384, 4096, 4096]

SPECS = [(4, 32), (32, 64), (64, 128), (128, 256), (256, 256),
         (384, 128), (192, 64), (96, 32), (64, 21)]
SPLITS = {5: (256, 128), 6: (128, 64), 7: (64, 32), 8: (32, 32)}
LEVEL_OF = [0, 1, 2, 3, 4, 3, 2, 1, 0]
FIRST_OF_LEVEL = [False, True, True, True, True, False, False, False, False]
BM = 512

_MM_CACHE = {}
_FIN_CACHE = {}
_SM_CACHE = {}
_GMM_CACHE = {}
_POOL_CACHE = {}
_UNPOOL_CACHE = {}


def _ceil_to(v, m):
    return (v + m - 1) // m * m


def _mm_call(np_rows, ca, cb, op):
    key = (np_rows, ca, cb, op)
    if key in _MM_CACHE:
        return _MM_CACHE[key]
    two = cb > 0

    if two:
        def body(a_ref, b_ref, ga_ref, gb_ref, ra_ref, rb_ref, bias_ref,
                 y_ref, r_ref):
            y = jnp.dot(a_ref[...], ga_ref[...],
                        preferred_element_type=jnp.float32)
            y += jnp.dot(b_ref[...], gb_ref[...],
                         preferred_element_type=jnp.float32)
            r = jnp.dot(a_ref[...], ra_ref[...],
                        preferred_element_type=jnp.float32)
            r += jnp.dot(b_ref[...], rb_ref[...],
                         preferred_element_type=jnp.float32)
            y_ref[...] = y
            r_ref[...] = r + bias_ref[0:1, :]
        in_specs = [
            pl.BlockSpec((BM, ca), lambda i: (i, 0)),
            pl.BlockSpec((BM, cb), lambda i: (i, 0)),
            pl.BlockSpec((ca, 3 * op), lambda i: (0, 0)),
            pl.BlockSpec((cb, 3 * op), lambda i: (0, 0)),
            pl.BlockSpec((ca, op), lambda i: (0, 0)),
            pl.BlockSpec((cb, op), lambda i: (0, 0)),
            pl.BlockSpec((8, op), lambda i: (0, 0)),
        ]
    else:
        def body(a_ref, ga_ref, ra_ref, bias_ref, y_ref, r_ref):
            y_ref[...] = jnp.dot(a_ref[...], ga_ref[...],
                                 preferred_element_type=jnp.float32)
            r = jnp.dot(a_ref[...], ra_ref[...],
                        preferred_element_type=jnp.float32)
            r_ref[...] = r + bias_ref[0:1, :]
        in_specs = [
            pl.BlockSpec((BM, ca), lambda i: (i, 0)),
            pl.BlockSpec((ca, 3 * op), lambda i: (0, 0)),
            pl.BlockSpec((ca, op), lambda i: (0, 0)),
            pl.BlockSpec((8, op), lambda i: (0, 0)),
        ]

    fn = pl.pallas_call(
        body,
        grid=(np_rows // BM,),
        in_specs=in_specs,
        out_specs=[pl.BlockSpec((BM, 3 * op), lambda i: (i, 0)),
                   pl.BlockSpec((BM, op), lambda i: (i, 0))],
        out_shape=[jax.ShapeDtypeStruct((np_rows, 3 * op), jnp.float32),
                   jax.ShapeDtypeStruct((np_rows, op), jnp.float32)],
    )
    _MM_CACHE[key] = fn
    return fn


def _fin_call(np_rows, op, accw, cntw, cntoff):
    key = (np_rows, op, accw, cntw, cntoff)
    if key in _FIN_CACHE:
        return _FIN_CACHE[key]

    def body(acc_ref, cnt_ref, r_ref, o_ref):
        s = acc_ref[0, :, :op] + acc_ref[1, :, :op]
        c = cnt_ref[0, :, cntoff:cntoff + 1] + cnt_ref[1, :, cntoff:cntoff + 1]
        o = s / jnp.maximum(c, 1.0) + r_ref[...]
        o_ref[...] = jnp.maximum(o, 0.0)

    fn = pl.pallas_call(
        body,
        grid=(np_rows // BM,),
        in_specs=[
            pl.BlockSpec((2, BM, accw), lambda i: (0, i, 0)),
            pl.BlockSpec((2, BM, cntw), lambda i: (0, i, 0)),
            pl.BlockSpec((BM, op), lambda i: (i, 0)),
        ],
        out_specs=pl.BlockSpec((BM, op), lambda i: (i, 0)),
        out_shape=jax.ShapeDtypeStruct((np_rows, op), jnp.float32),
    )
    _FIN_CACHE[key] = fn
    return fn


def _softmax_call(np_rows, op, accw, cntw, cntoff, real_cols):
    key = (np_rows, op, accw, cntw, cntoff, real_cols)
    if key in _SM_CACHE:
        return _SM_CACHE[key]

    def body(acc_ref, cnt_ref, r_ref, o_ref):
        s = acc_ref[0, :, :op] + acc_ref[1, :, :op]
        c = cnt_ref[0, :, cntoff:cntoff + 1] + cnt_ref[1, :, cntoff:cntoff + 1]
        z = s / jnp.maximum(c, 1.0) + r_ref[...]
        col = lax.broadcasted_iota(jnp.int32, z.shape, 1)
        mask = col < real_cols
        zm = jnp.where(mask, z, -jnp.inf)
        m = jnp.max(zm, axis=1, keepdims=True)
        e = jnp.where(mask, jnp.exp(z - m), 0.0)
        o_ref[...] = e / jnp.sum(e, axis=1, keepdims=True)

    fn = pl.pallas_call(
        body,
        grid=(np_rows // BM,),
        in_specs=[
            pl.BlockSpec((2, BM, accw), lambda i: (0, i, 0)),
            pl.BlockSpec((2, BM, cntw), lambda i: (0, i, 0)),
            pl.BlockSpec((BM, op), lambda i: (i, 0)),
        ],
        out_specs=pl.BlockSpec((BM, op), lambda i: (i, 0)),
        out_shape=jax.ShapeDtypeStruct((np_rows, op), jnp.float32),
    )
    _SM_CACHE[key] = fn
    return fn


def _mesh():
    return plsc.VectorSubcoreMesh(core_axis_name="c", subcore_axis_name="s")


_SC_PARAMS = pltpu.CompilerParams(use_tc_tiling_on_sc=False,
                                  needs_layout_passes=False)


def _lane_consts():
    return [jnp.full((LN, 1), i, dtype=jnp.int32) for i in range(LN)]


_GDN = lax.GatherDimensionNumbers(
    offset_dims=(), collapsed_slice_dims=(0,), start_index_map=(0,))


def _bcast_lane(vec, lane_const):
    return lax.gather(vec, lane_const, _GDN, (1,),
                      mode=lax.GatherScatterMode.PROMISE_IN_BOUNDS)


def _gmm_sc_call(npad, epad, op, bsz, with_cnt):
    key = (npad, epad, op, bsz, with_cnt)
    if key in _GMM_CACHE:
        return _GMM_CACHE[key]
    cols = 3 * op
    opc = op + 16 if with_cnt else op
    nj = op // LN
    npt = epad // NW
    n_chunks = npt // bsz
    zrows = 16
    rpt_sc = npad // NS
    nz = rpt_sc // zrows

    @functools.partial(
        pl.kernel,
        out_type=jax.ShapeDtypeStruct((NC, npad, opc), jnp.float32),
        mesh=_mesh(),
        compiler_params=_SC_PARAMS,
        scratch_types=[
            pltpu.VMEM((n_chunks, bsz), jnp.int32),
            pltpu.VMEM((n_chunks, bsz), jnp.int32),
            pltpu.VMEM((2, bsz), jnp.float32),
            pltpu.VMEM((2, bsz), jnp.float32),
            pltpu.VMEM((KM, bsz), jnp.float32),
            pltpu.VMEM((bsz, cols), jnp.float32),
            pltpu.VMEM((bsz, cols), jnp.float32),
            pltpu.VMEM((bsz, opc), jnp.float32),
            pltpu.VMEM((KM, 2, LN), jnp.float32),
            pltpu.VMEM((KM, 2, LN), jnp.float32),
            pltpu.VMEM((zrows, opc), jnp.float32),
            pltpu.VMEM_SHARED((npad, opc), jnp.float32),
            pltpu.SemaphoreType.DMA,
            pltpu.SemaphoreType.DMA,
        ],
    )
    def k(src3, dst3, psR, s2e_h, mu_h, ytab, out_h,
          src_v, dst_v, ps0, ps1, w_v, rows0, rows1, msg_v, s2_v, mu_v,
          zero_v, acc_sh, sem, sem_ps):
        cid = lax.axis_index("c")
        sid = lax.axis_index("s")
        wid = sid * NC + cid
        lanes = _lane_consts()
        zv = jnp.zeros((LN,), jnp.float32)
        onev = jnp.where(lax.iota(jnp.int32, LN) == 0, 1.0, 0.0)
        rows_v = [rows0, rows1]
        ps_v = [ps0, ps1]
        nzc = opc // LN

        pltpu.sync_copy(src3.at[wid], src_v)
        pltpu.sync_copy(dst3.at[wid], dst_v)
        pltpu.sync_copy(s2e_h, s2_v)
        pltpu.sync_copy(mu_h, mu_v)
        pltpu.async_copy(ytab.at[src_v.at[0]], rows_v[0], sem)
        pltpu.async_copy(psR.at[wid, 0], ps_v[0], sem_ps)

        for r in range(zrows):
            for j in range(nzc):
                zero_v[r, pl.ds(j * LN, LN)] = zv
        if with_cnt:
            for r in range(bsz):
                msg_v[r, pl.ds(op, LN)] = onev
        zbase = sid * rpt_sc

        @pl.loop(0, nz)
        def _zero(i):
            pltpu.sync_copy(zero_v, acc_sh.at[pl.ds(zbase + i * zrows, zrows)])

        plsc.subcore_barrier()

        def chunk_body(i, b):
            inext = jnp.where(i + 1 < n_chunks, i + 1, 0)
            pltpu.async_copy(ytab.at[src_v.at[inext]], rows_v[1 - b], sem)
            pltpu.async_copy(psR.at[wid, inext], ps_v[1 - b], sem_ps)
            pltpu.make_async_copy(psR.at[wid, 0], ps_v[b], sem_ps).wait()
            pltpu.make_async_copy(ytab.at[src_v.at[0]], rows_v[b], sem).wait()
            rows = rows_v[b]
            ps = ps_v[b]
            for t in range(bsz // LN):
                p0 = ps[0, pl.ds(t * LN, LN)]
                p1 = ps[1, pl.ds(t * LN, LN)]
                for kk in range(KM):
                    d0 = p0 - mu_v[kk, 0]
                    d1 = p1 - mu_v[kk, 1]
                    q = d0 * d0 / s2_v[kk, 0] + d1 * d1 / s2_v[kk, 1]
                    w_v[kk, pl.ds(t * LN, LN)] = jnp.exp(q * (-0.5))
            for t in range(bsz // LN):
                w0 = w_v[0, pl.ds(t * LN, LN)]
                w1 = w_v[1, pl.ds(t * LN, LN)]
                w2 = w_v[2, pl.ds(t * LN, LN)]
                for ln in range(LN):
                    e = t * LN + ln
                    b0 = _bcast_lane(w0, lanes[ln])
                    b1 = _bcast_lane(w1, lanes[ln])
                    b2 = _bcast_lane(w2, lanes[ln])
                    for j in range(nj):
                        m = (b0 * rows[e, pl.ds(j * LN, LN)]
                             + b1 * rows[e, pl.ds(op + j * LN, LN)]
                             + b2 * rows[e, pl.ds(2 * op + j * LN, LN)])
                        msg_v[e, pl.ds(j * LN, LN)] = m
            pltpu.sync_copy(msg_v, acc_sh.at[dst_v.at[i]], add=True)

        @pl.loop(0, n_chunks // 2)
        def _pair(ii):
            chunk_body(ii * 2, 0)
            chunk_body(ii * 2 + 1, 1)

        if n_chunks % 2:
            chunk_body(n_chunks - 1, (n_chunks - 1) % 2)
        pltpu.make_async_copy(ytab.at[src_v.at[0]], rows_v[n_chunks % 2],
                              sem).wait()
        pltpu.make_async_copy(psR.at[wid, 0], ps_v[n_chunks % 2],
                              sem_ps).wait()

        plsc.subcore_barrier()

        @pl.loop(0, nz)
        def _out(i):
            b = zbase + i * zrows
            pltpu.sync_copy(acc_sh.at[pl.ds(b, zrows)],
                            out_h.at[cid, pl.ds(b, zrows)])

    _GMM_CACHE[key] = k
    return k


_CNT_CACHE = {}


def _cnt_sc_call(npad, epad, bsz):
    key = (npad, epad, bsz)
    if key in _CNT_CACHE:
        return _CNT_CACHE[key]
    npt = epad // NW
    n_chunks = npt // bsz
    zrows = 16
    rpt_sc = npad // NS
    nz = rpt_sc // zrows

    @functools.partial(
        pl.kernel,
        out_type=jax.ShapeDtypeStruct((NC, npad, 16), jnp.float32),
        mesh=_mesh(),
        compiler_params=_SC_PARAMS,
        scratch_types=[
            pltpu.VMEM((n_chunks, bsz), jnp.int32),
            pltpu.VMEM((bsz, 16), jnp.float32),
            pltpu.VMEM((zrows, 16), jnp.float32),
            pltpu.VMEM_SHARED((npad, 16), jnp.float32),
        ],
    )
    def k(dst3, out_h, dst_v, ones_v, zero_v, acc_sh):
        cid = lax.axis_index("c")
        sid = lax.axis_index("s")
        wid = sid * NC + cid
        zv = jnp.zeros((LN,), jnp.float32)
        ov = jnp.where(lax.iota(jnp.int32, LN) == 0, 1.0, 0.0)
        pltpu.sync_copy(dst3.at[wid], dst_v)
        for r in range(zrows):
            zero_v[r, pl.ds(0, LN)] = zv
        for r in range(bsz):
            ones_v[r, pl.ds(0, LN)] = ov
        zbase = sid * rpt_sc

        @pl.loop(0, nz)
        def _zero(i):
            pltpu.sync_copy(zero_v, acc_sh.at[pl.ds(zbase + i * zrows, zrows)])

        plsc.subcore_barrier()

        @pl.loop(0, n_chunks)
        def _chunk(i):
            pltpu.sync_copy(ones_v, acc_sh.at[dst_v.at[i]], add=True)

        plsc.subcore_barrier()

        @pl.loop(0, nz)
        def _out(i):
            b = zbase + i * zrows
            pltpu.sync_copy(acc_sh.at[pl.ds(b, zrows)],
                            out_h.at[cid, pl.ds(b, zrows)])

    _CNT_CACHE[key] = k
    return k


def _pool_sc_call(np_coarse, cch, l_real, big):
    key = (np_coarse, cch, l_real)
    if key in _POOL_CACHE:
        return _POOL_CACHE[key]
    nj = cch // LN
    rpt = np_coarse // NW

    @functools.partial(
        pl.kernel,
        out_type=(jax.ShapeDtypeStruct((np_coarse, cch), jnp.float32),
                  jax.ShapeDtypeStruct((np_coarse, cch), jnp.int32)),
        mesh=_mesh(),
        compiler_params=_SC_PARAMS,
        scratch_types=[
            pltpu.VMEM((7, rpt), jnp.int32),
            pltpu.VMEM((7, LN, cch), jnp.float32),
            pltpu.VMEM((LN, cch), jnp.float32),
            pltpu.VMEM((LN, cch), jnp.int32),
            pltpu.SemaphoreType.DMA,
        ],
    )
    def k(x_h, hexT, vals_h, idx_h, hx_v, xg_v, vb_v, ib_v, sem):
        cid = lax.axis_index("c")
        sid = lax.axis_index("s")
        wid = sid * NC + cid
        lanes = _lane_consts()
        bigv = jnp.full((LN,), big, dtype=jnp.int32)
        row0 = wid * rpt
        for j in range(7):
            pltpu.sync_copy(hexT.at[j, pl.ds(row0, rpt)], hx_v.at[j])

        @pl.loop(0, rpt // LN)
        def _chunk(ch):
            for j in range(7):
                pltpu.async_copy(x_h.at[hx_v.at[j, pl.ds(ch * LN, LN)]],
                                 xg_v.at[j], sem)
            for j in range(7):
                pltpu.make_async_copy(x_h.at[hx_v.at[0, pl.ds(0, LN)]],
                                      xg_v.at[j], sem).wait()
            base = row0 + ch * LN
            for r in range(LN):
                hv0 = hx_v[0, pl.ds(ch * LN, LN)]
                bi0 = _bcast_lane(hv0, lanes[r])
                bv = [xg_v[0, r, pl.ds(j * LN, LN)] for j in range(nj)]
                bi = [bi0 for _ in range(nj)]
                for j in range(1, 7):
                    hv = hx_v[j, pl.ds(ch * LN, LN)]
                    bj = _bcast_lane(hv, lanes[r])
                    for jc in range(nj):
                        v = xg_v[j, r, pl.ds(jc * LN, LN)]
                        m = v > bv[jc]
                        bv[jc] = jnp.where(m, v, bv[jc])
                        bi[jc] = jnp.where(m, bj, bi[jc])
                valid = jnp.full((LN,), base + r < l_real)
                for jc in range(nj):
                    vb_v[r, pl.ds(jc * LN, LN)] = bv[jc]
                    ib_v[r, pl.ds(jc * LN, LN)] = jnp.where(valid, bi[jc],
                                                            bigv)
            pltpu.sync_copy(vb_v, vals_h.at[pl.ds(base, LN)])
            pltpu.sync_copy(ib_v, idx_h.at[pl.ds(base, LN)])

    _POOL_CACHE[key] = k
    return k


def _unpool_sc_call(np_coarse, np_fine, cch, bu):
    key = (np_coarse, np_fine, cch, bu)
    if key in _UNPOOL_CACHE:
        return _UNPOOL_CACHE[key]
    nj = cch // LN
    rt = np_fine // NW
    n_chunks = np_coarse // bu

    @functools.partial(
        pl.kernel,
        out_type=jax.ShapeDtypeStruct((np_fine, cch), jnp.float32),
        mesh=_mesh(),
        compiler_params=_SC_PARAMS,
        scratch_types=[
            pltpu.VMEM((bu, cch), jnp.float32),
            pltpu.VMEM((bu, cch), jnp.float32),
            pltpu.VMEM((bu, cch), jnp.int32),
            pltpu.VMEM((bu, cch), jnp.int32),
            pltpu.VMEM((rt, cch), jnp.float32),
            pltpu.SemaphoreType.DMA,
            pltpu.SemaphoreType.DMA,
        ],
    )
    def k(vals_h, idx_h, y_h, v0, v1, i0, i1, y_v, sem_v, sem_i):
        cid = lax.axis_index("c")
        sid = lax.axis_index("s")
        wid = sid * NC + cid
        base = wid * rt
        basev = jnp.full((LN,), 0, dtype=jnp.int32) + base
        rtv = jnp.full((LN,), rt, dtype=jnp.int32)
        zv = jnp.zeros((LN,), jnp.float32)
        cols = [lax.iota(jnp.int32, LN) + (jc * LN) for jc in range(nj)]
        vbuf = [v0, v1]
        ibuf = [i0, i1]

        pltpu.async_copy(vals_h.at[pl.ds(0, bu)], v0, sem_v)
        pltpu.async_copy(idx_h.at[pl.ds(0, bu)], i0, sem_i)

        @pl.loop(0, rt)
        def _zero(r):
            for j in range(nj):
                y_v[r, pl.ds(j * LN, LN)] = zv

        def chunk_body(i, b):
            inext = jnp.where(i + 1 < n_chunks, i + 1, 0)
            pltpu.async_copy(vals_h.at[pl.ds(inext * bu, bu)], vbuf[1 - b],
                             sem_v)
            pltpu.async_copy(idx_h.at[pl.ds(inext * bu, bu)], ibuf[1 - b],
                             sem_i)
            pltpu.make_async_copy(vals_h.at[pl.ds(0, bu)], vbuf[b],
                                  sem_v).wait()
            pltpu.make_async_copy(idx_h.at[pl.ds(0, bu)], ibuf[b],
                                  sem_i).wait()
            for r in range(bu):
                for jc in range(nj):
                    iv = ibuf[b][r, pl.ds(jc * LN, LN)]
                    vv = vbuf[b][r, pl.ds(jc * LN, LN)]
                    lr = iv - basev
                    m = lr.astype(jnp.uint32) < rtv.astype(jnp.uint32)
                    plsc.store_scatter(y_v, [lr, cols[jc]], vv, mask=m)

        @pl.loop(0, n_chunks // 2)
        def _pair(ii):
            chunk_body(ii * 2, 0)
            chunk_body(ii * 2 + 1, 1)

        if n_chunks % 2:
            chunk_body(n_chunks - 1, (n_chunks - 1) % 2)
        pltpu.make_async_copy(vals_h.at[pl.ds(0, bu)], vbuf[n_chunks % 2],
                              sem_v).wait()
        pltpu.make_async_copy(idx_h.at[pl.ds(0, bu)], ibuf[n_chunks % 2],
                              sem_i).wait()

        pltpu.sync_copy(y_v, y_h.at[pl.ds(base, rt)])

    _UNPOOL_CACHE[key] = k
    return k


def _pad_rows(a, rows):
    return jnp.pad(a, ((0, rows - a.shape[0]), (0, 0)))


def _prep_conv(p, ci):
    g, mu, sigma, root, bias = p
    in_c, out_c = SPECS[ci]
    op = _ceil_to(out_c, 32)
    g3 = g.reshape(in_c, KM, out_c)
    g3 = jnp.pad(g3, ((0, 0), (0, 0), (0, op - out_c)))
    gp = g3.reshape(in_c, KM * op)
    rootp = jnp.pad(root, ((0, 0), (0, op - out_c)))
    biasp = jnp.pad(bias, (0, op - out_c))
    bias8 = jnp.broadcast_to(biasp[None, :], (8, op))
    s2e = jnp.broadcast_to((sigma ** 2 + 1e-16)[:, :, None], (KM, 2, LN))
    mub = jnp.broadcast_to(mu[:, :, None], (KM, 2, LN))
    if ci == 0:
        gp = jnp.pad(gp, ((0, 4), (0, 0)))
        rootp = jnp.pad(rootp, ((0, 4), (0, 0)))
    return gp, rootp, bias8, s2e, mub, op


def _gmm_bsz(op):
    return {32: 64, 64: 64, 128: 32, 256: 16}[op]


def kernel(x, edge_index, edges_coarse, pseudos, hexes, params):
    eis, psTs = [], []
    for lv in range(5):
        ei = edge_index if lv == 0 else edges_coarse[lv - 1]
        ep = EPAD[lv]
        pad = ep - EV[lv]
        src = jnp.pad(ei[0], (0, pad))
        dst = jnp.pad(ei[1], (0, pad), constant_values=NV[lv])
        eis.append((src, dst))
        psTs.append(jnp.pad(pseudos[lv].T, ((0, 0), (0, pad))))
    hexTs = []
    for lv in range(4):
        lc = NV[lv + 1]
        hexT = hexes[lv][:lc].T
        hexTs.append(jnp.pad(hexT, ((0, 0), (0, NPAD[lv + 1] - lc))))
    prep = [_prep_conv(params[i], i) for i in range(9)]

    level_cnt_acc = {}
    dst3_cnt = eis[0][1].reshape(NW, EPAD[0] // NW // 128, 128)
    cnt0 = _cnt_sc_call(NPAD[0], EPAD[0], 128)(dst3_cnt)
    level_cnt_acc[0] = (cnt0, 16, 0)

    def gmm(ci, xa, xb, relu):
        lv = LEVEL_OF[ci]
        gp, rootp, bias8, s2e, mub, op = prep[ci]
        npr = NPAD[lv]
        if xb is None:
            y, r = _mm_call(npr, xa.shape[1], 0, op)(xa, gp, rootp, bias8)
        else:
            ca, cb = SPLITS[ci]
            y, r = _mm_call(npr, ca, cb, op)(
                xa, xb, gp[:ca], gp[ca:], rootp[:ca], rootp[ca:], bias8)
        bsz = _gmm_bsz(op)
        npt = EPAD[lv] // NW
        nch = npt // bsz
        src3 = eis[lv][0].reshape(NW, nch, bsz)
        dst3 = eis[lv][1].reshape(NW, nch, bsz)
        psR = psTs[lv].reshape(2, NW, nch, bsz).transpose(1, 2, 0, 3)
        first = FIRST_OF_LEVEL[ci]
        acc = _gmm_sc_call(npr, EPAD[lv], op, bsz, first)(
            src3, dst3, psR, s2e, mub, y)
        if first:
            level_cnt_acc[lv] = (acc, op + 16, op)
        cnt_acc, cntw, cntoff = level_cnt_acc[lv]
        accw = op + 16 if first else op
        if relu:
            return _fin_call(npr, op, accw, cntw, cntoff)(acc, cnt_acc, r)
        return _softmax_call(npr, op, accw, cntw, cntoff, SPECS[ci][1])(
            acc, cnt_acc, r)

    def pool(lv, xv):
        cch = xv.shape[1]
        return _pool_sc_call(NPAD[lv + 1], cch, NV[lv + 1], 1 << 28)(
            xv, hexTs[lv])

    def unpool(lv, vals, idx):
        cch = vals.shape[1]
        bu = {256: 16, 128: 32, 64: 64, 32: 128}[cch]
        return _unpool_sc_call(NPAD[lv + 1], NPAD[lv], cch, bu)(vals, idx)

    x0in = _pad_rows(jnp.pad(x, ((0, 0), (0, 4))), NPAD[0])
    x0 = gmm(0, x0in, None, True)
    x1, i1 = pool(0, x0)
    h = gmm(1, x1, None, True)
    x2, i2 = pool(1, h)
    h = gmm(2, x2, None, True)
    x3, i3 = pool(2, h)
    h = gmm(3, x3, None, True)
    x4, i4 = pool(3, h)
    h = gmm(4, x4, None, True)
    h = unpool(3, h, i4)
    h = gmm(5, h, x3, True)
    h = unpool(2, h, i3)
    h = gmm(6, h, x2, True)
    h = unpool(1, h, i2)
    h = gmm(7, h, x1, True)
    h = unpool(0, h, i1)
    out = gmm(8, h, x0, False)
    return out[:NV[0], :SPECS[8][1]]
